# initial kernel scaffold (unmeasured)
import jax
import jax.numpy as jnp
from jax import lax
from jax.experimental import pallas as pl
from jax.experimental.pallas import tpu as pltpu


def kernel(
    x,
):
    def body(*refs):
        pass

    out_shape = jax.ShapeDtypeStruct(..., jnp.float32)
    return pl.pallas_call(body, out_shape=out_shape)(...)


# baseline (device time: 194354 ns/iter reference)
import jax
import jax.numpy as jnp
from jax import lax
from jax.experimental import pallas as pl
from jax.experimental.pallas import tpu as pltpu

M, N = 2048, 1024


def kernel(x):
    x = x.reshape(M, N)

    def body(x_ref, out_ref, recv1, recv2, send_sems, recv_sems):
        my_x = lax.axis_index("x")
        my_y = lax.axis_index("y")
        y_nbr = (my_x, 1 - my_y)
        x_nbr = (1 - my_x, my_y)

        barrier_sem = pltpu.get_barrier_semaphore()
        for nbr in (y_nbr, x_nbr):
            pl.semaphore_signal(
                barrier_sem, inc=1,
                device_id=nbr, device_id_type=pl.DeviceIdType.MESH,
            )
        pl.semaphore_wait(barrier_sem, 2)

        rdma1 = pltpu.make_async_remote_copy(
            src_ref=x_ref,
            dst_ref=recv1,
            send_sem=send_sems.at[0],
            recv_sem=recv_sems.at[0],
            device_id=y_nbr,
            device_id_type=pl.DeviceIdType.MESH,
        )
        rdma1.start()
        rdma1.wait()
        out_ref[...] = x_ref[...] + recv1[...]

        rdma2 = pltpu.make_async_remote_copy(
            src_ref=out_ref,
            dst_ref=recv2,
            send_sem=send_sems.at[1],
            recv_sem=recv_sems.at[1],
            device_id=x_nbr,
            device_id_type=pl.DeviceIdType.MESH,
        )
        rdma2.start()
        rdma2.wait()
        out_ref[...] = out_ref[...] + recv2[...]

    return pl.pallas_call(
        body,
        out_shape=jax.ShapeDtypeStruct((M, N), jnp.float32),
        in_specs=[pl.BlockSpec(memory_space=pltpu.VMEM)],
        out_specs=pl.BlockSpec(memory_space=pltpu.VMEM),
        scratch_shapes=[
            pltpu.VMEM((M, N), jnp.float32),
            pltpu.VMEM((M, N), jnp.float32),
            pltpu.SemaphoreType.DMA((2,)),
            pltpu.SemaphoreType.DMA((2,)),
        ],
        compiler_params=pltpu.CompilerParams(collective_id=0),
    )(x)


# device time: 84318 ns/iter; 2.3050x vs baseline; 2.3050x over previous
import jax
import jax.numpy as jnp
from jax import lax
from jax.experimental import pallas as pl
from jax.experimental.pallas import tpu as pltpu

M, N = 2048, 1024
H = 512
Q = 256


def kernel(x):
    x = x.reshape(M, N)

    def body(x_ref, out_ref, rA0, rA1, rB0, rB1, send_sems, recv_sems):
        my_x = lax.axis_index("x")
        my_y = lax.axis_index("y")
        y_nbr = (my_x, 1 - my_y)
        x_nbr = (1 - my_x, my_y)

        barrier_sem = pltpu.get_barrier_semaphore()
        for nbr in (y_nbr, x_nbr):
            pl.semaphore_signal(
                barrier_sem, inc=1,
                device_id=nbr, device_id_type=pl.DeviceIdType.MESH,
            )
        pl.semaphore_wait(barrier_sem, 2)

        def xfer(i, src, dst, nbr):
            return pltpu.make_async_remote_copy(
                src_ref=src, dst_ref=dst,
                send_sem=send_sems.at[i], recv_sem=recv_sems.at[i],
                device_id=nbr, device_id_type=pl.DeviceIdType.MESH,
            )

        aH_mine = my_y * H
        aH_peer = (1 - my_y) * H
        qA = aH_mine + my_x * Q
        qA_peer = aH_mine + (1 - my_x) * Q
        bH_mine = 1024 + my_x * H
        bH_peer = 1024 + (1 - my_x) * H
        qB = bH_mine + my_y * Q
        qB_peer = bH_mine + (1 - my_y) * Q

        a0 = xfer(0, x_ref.at[pl.ds(aH_peer, H), :], rA0, y_nbr)
        b0 = xfer(4, x_ref.at[pl.ds(bH_peer, H), :], rB0, x_nbr)
        a0.start()
        b0.start()
        a0.wait()
        out_ref[pl.ds(aH_mine, H), :] = x_ref[pl.ds(aH_mine, H), :] + rA0[...]
        b0.wait()
        out_ref[pl.ds(bH_mine, H), :] = x_ref[pl.ds(bH_mine, H), :] + rB0[...]

        a1 = xfer(1, out_ref.at[pl.ds(qA_peer, Q), :], rA1, x_nbr)
        b1 = xfer(5, out_ref.at[pl.ds(qB_peer, Q), :], rB1, y_nbr)
        a1.start()
        b1.start()
        a1.wait()
        out_ref[pl.ds(qA, Q), :] = out_ref[pl.ds(qA, Q), :] + rA1[...]
        b1.wait()
        out_ref[pl.ds(qB, Q), :] = out_ref[pl.ds(qB, Q), :] + rB1[...]

        a2 = xfer(2, out_ref.at[pl.ds(qA, Q), :],
                  out_ref.at[pl.ds(qA, Q), :], x_nbr)
        b2 = xfer(6, out_ref.at[pl.ds(qB, Q), :],
                  out_ref.at[pl.ds(qB, Q), :], y_nbr)
        a2.start()
        b2.start()
        a2.wait()
        b2.wait()

        a3 = xfer(3, out_ref.at[pl.ds(aH_mine, H), :],
                  out_ref.at[pl.ds(aH_mine, H), :], y_nbr)
        b3 = xfer(7, out_ref.at[pl.ds(bH_mine, H), :],
                  out_ref.at[pl.ds(bH_mine, H), :], x_nbr)
        a3.start()
        b3.start()
        a3.wait()
        b3.wait()

    return pl.pallas_call(
        body,
        out_shape=jax.ShapeDtypeStruct((M, N), jnp.float32),
        in_specs=[pl.BlockSpec(memory_space=pltpu.VMEM)],
        out_specs=pl.BlockSpec(memory_space=pltpu.VMEM),
        scratch_shapes=[
            pltpu.VMEM((H, N), jnp.float32),
            pltpu.VMEM((Q, N), jnp.float32),
            pltpu.VMEM((H, N), jnp.float32),
            pltpu.VMEM((Q, N), jnp.float32),
            pltpu.SemaphoreType.DMA((8,)),
            pltpu.SemaphoreType.DMA((8,)),
        ],
        compiler_params=pltpu.CompilerParams(collective_id=0),
    )(x)


# device time: 82685 ns/iter; 2.3505x vs baseline; 1.0197x over previous
import jax
import jax.numpy as jnp
from jax import lax
from jax.experimental import pallas as pl
from jax.experimental.pallas import tpu as pltpu

M, N = 2048, 1024
H = 512
Q = 256


def kernel(x):
    x = x.reshape(M, N)

    def body(x_ref, out_ref, rA0, rA1, rB0, rB1, send_sems, recv_sems):
        my_x = lax.axis_index("x")
        my_y = lax.axis_index("y")
        y_nbr = (my_x, 1 - my_y)
        x_nbr = (1 - my_x, my_y)

        barrier_sem = pltpu.get_barrier_semaphore()
        for nbr in (y_nbr, x_nbr):
            pl.semaphore_signal(
                barrier_sem, inc=1,
                device_id=nbr, device_id_type=pl.DeviceIdType.MESH,
            )
        pl.semaphore_wait(barrier_sem, 2)

        def xfer(i, src, dst, nbr):
            return pltpu.make_async_remote_copy(
                src_ref=src, dst_ref=dst,
                send_sem=send_sems.at[i], recv_sem=recv_sems.at[i],
                device_id=nbr, device_id_type=pl.DeviceIdType.MESH,
            )

        aH_mine = my_y * H
        aH_peer = (1 - my_y) * H
        qA = aH_mine + my_x * Q
        qA_peer = aH_mine + (1 - my_x) * Q
        bH_mine = 1024 + my_x * H
        bH_peer = 1024 + (1 - my_x) * H
        qB = bH_mine + my_y * Q
        qB_peer = bH_mine + (1 - my_y) * Q
        fA = (1 - my_x) * Q
        sA = my_x * Q
        fB = (1 - my_y) * Q
        sB = my_y * Q

        a0f = xfer(0, x_ref.at[pl.ds(aH_peer + fA, Q), :],
                   rA0.at[pl.ds(fA, Q), :], y_nbr)
        a0s = xfer(1, x_ref.at[pl.ds(aH_peer + sA, Q), :],
                   rA0.at[pl.ds(sA, Q), :], y_nbr)
        b0f = xfer(2, x_ref.at[pl.ds(bH_peer + fB, Q), :],
                   rB0.at[pl.ds(fB, Q), :], x_nbr)
        b0s = xfer(3, x_ref.at[pl.ds(bH_peer + sB, Q), :],
                   rB0.at[pl.ds(sB, Q), :], x_nbr)
        a0f.start()
        a0s.start()
        b0f.start()
        b0s.start()

        a1 = xfer(4, out_ref.at[pl.ds(qA_peer, Q), :], rA1, x_nbr)
        b1 = xfer(5, out_ref.at[pl.ds(qB_peer, Q), :], rB1, y_nbr)
        a0f.wait_recv()
        out_ref[pl.ds(qA_peer, Q), :] = (
            x_ref[pl.ds(qA_peer, Q), :] + rA0[pl.ds(fA, Q), :]
        )
        a1.start()
        b0f.wait_recv()
        out_ref[pl.ds(qB_peer, Q), :] = (
            x_ref[pl.ds(qB_peer, Q), :] + rB0[pl.ds(fB, Q), :]
        )
        b1.start()

        a0s.wait_recv()
        out_ref[pl.ds(qA, Q), :] = (
            x_ref[pl.ds(qA, Q), :] + rA0[pl.ds(sA, Q), :]
        )
        b0s.wait_recv()
        out_ref[pl.ds(qB, Q), :] = (
            x_ref[pl.ds(qB, Q), :] + rB0[pl.ds(sB, Q), :]
        )

        a2 = xfer(6, out_ref.at[pl.ds(qA, Q), :],
                  out_ref.at[pl.ds(qA, Q), :], x_nbr)
        b2 = xfer(7, out_ref.at[pl.ds(qB, Q), :],
                  out_ref.at[pl.ds(qB, Q), :], y_nbr)
        a1.wait_recv()
        out_ref[pl.ds(qA, Q), :] = out_ref[pl.ds(qA, Q), :] + rA1[...]
        a2.start()
        b1.wait_recv()
        out_ref[pl.ds(qB, Q), :] = out_ref[pl.ds(qB, Q), :] + rB1[...]
        b2.start()

        a3 = xfer(8, out_ref.at[pl.ds(aH_mine, H), :],
                  out_ref.at[pl.ds(aH_mine, H), :], y_nbr)
        b3 = xfer(9, out_ref.at[pl.ds(bH_mine, H), :],
                  out_ref.at[pl.ds(bH_mine, H), :], x_nbr)
        a2.wait_recv()
        a3.start()
        b2.wait_recv()
        b3.start()

        a3.wait_recv()
        b3.wait_recv()
        for r in (a0f, a0s, b0f, b0s, a1, b1, a2, b2, a3, b3):
            r.wait_send()

    return pl.pallas_call(
        body,
        out_shape=jax.ShapeDtypeStruct((M, N), jnp.float32),
        in_specs=[pl.BlockSpec(memory_space=pltpu.VMEM)],
        out_specs=pl.BlockSpec(memory_space=pltpu.VMEM),
        scratch_shapes=[
            pltpu.VMEM((H, N), jnp.float32),
            pltpu.VMEM((Q, N), jnp.float32),
            pltpu.VMEM((H, N), jnp.float32),
            pltpu.VMEM((Q, N), jnp.float32),
            pltpu.SemaphoreType.DMA((10,)),
            pltpu.SemaphoreType.DMA((10,)),
        ],
        compiler_params=pltpu.CompilerParams(collective_id=0),
    )(x)


# device time: 82673 ns/iter; 2.3509x vs baseline; 1.0001x over previous
import jax
import jax.numpy as jnp
from jax import lax
from jax.experimental import pallas as pl
from jax.experimental.pallas import tpu as pltpu

M, N = 2048, 1024
H = 512
Q = 256


def kernel(x):
    def body(x4_ref, out_ref, rA0, rA1, rB0, rB1, send_sems, recv_sems):
        x_ref = x4_ref.at[0, 0]
        my_x = lax.axis_index("x")
        my_y = lax.axis_index("y")
        y_nbr = (my_x, 1 - my_y)
        x_nbr = (1 - my_x, my_y)

        barrier_sem = pltpu.get_barrier_semaphore()
        for nbr in (y_nbr, x_nbr):
            pl.semaphore_signal(
                barrier_sem, inc=1,
                device_id=nbr, device_id_type=pl.DeviceIdType.MESH,
            )
        pl.semaphore_wait(barrier_sem, 2)

        def xfer(i, src, dst, nbr):
            return pltpu.make_async_remote_copy(
                src_ref=src, dst_ref=dst,
                send_sem=send_sems.at[i], recv_sem=recv_sems.at[i],
                device_id=nbr, device_id_type=pl.DeviceIdType.MESH,
            )

        aH_mine = my_y * H
        aH_peer = (1 - my_y) * H
        qA = aH_mine + my_x * Q
        qA_peer = aH_mine + (1 - my_x) * Q
        bH_mine = 1024 + my_x * H
        bH_peer = 1024 + (1 - my_x) * H
        qB = bH_mine + my_y * Q
        qB_peer = bH_mine + (1 - my_y) * Q
        fA = (1 - my_x) * Q
        sA = my_x * Q
        fB = (1 - my_y) * Q
        sB = my_y * Q

        a0f = xfer(0, x_ref.at[pl.ds(aH_peer + fA, Q), :],
                   rA0.at[pl.ds(fA, Q), :], y_nbr)
        a0s = xfer(1, x_ref.at[pl.ds(aH_peer + sA, Q), :],
                   rA0.at[pl.ds(sA, Q), :], y_nbr)
        b0f = xfer(2, x_ref.at[pl.ds(bH_peer + fB, Q), :],
                   rB0.at[pl.ds(fB, Q), :], x_nbr)
        b0s = xfer(3, x_ref.at[pl.ds(bH_peer + sB, Q), :],
                   rB0.at[pl.ds(sB, Q), :], x_nbr)
        a0f.start()
        a0s.start()
        b0f.start()
        b0s.start()

        a1 = xfer(4, out_ref.at[pl.ds(qA_peer, Q), :], rA1, x_nbr)
        b1 = xfer(5, out_ref.at[pl.ds(qB_peer, Q), :], rB1, y_nbr)
        a0f.wait_recv()
        out_ref[pl.ds(qA_peer, Q), :] = (
            x_ref[pl.ds(qA_peer, Q), :] + rA0[pl.ds(fA, Q), :]
        )
        a1.start()
        b0f.wait_recv()
        out_ref[pl.ds(qB_peer, Q), :] = (
            x_ref[pl.ds(qB_peer, Q), :] + rB0[pl.ds(fB, Q), :]
        )
        b1.start()

        a0s.wait_recv()
        out_ref[pl.ds(qA, Q), :] = (
            x_ref[pl.ds(qA, Q), :] + rA0[pl.ds(sA, Q), :]
        )
        b0s.wait_recv()
        out_ref[pl.ds(qB, Q), :] = (
            x_ref[pl.ds(qB, Q), :] + rB0[pl.ds(sB, Q), :]
        )

        a2 = xfer(6, out_ref.at[pl.ds(qA, Q), :],
                  out_ref.at[pl.ds(qA, Q), :], x_nbr)
        b2 = xfer(7, out_ref.at[pl.ds(qB, Q), :],
                  out_ref.at[pl.ds(qB, Q), :], y_nbr)
        a1.wait_recv()
        out_ref[pl.ds(qA, Q), :] = out_ref[pl.ds(qA, Q), :] + rA1[...]
        a2.start()
        b1.wait_recv()
        out_ref[pl.ds(qB, Q), :] = out_ref[pl.ds(qB, Q), :] + rB1[...]
        b2.start()

        a3 = xfer(8, out_ref.at[pl.ds(aH_mine, H), :],
                  out_ref.at[pl.ds(aH_mine, H), :], y_nbr)
        b3 = xfer(9, out_ref.at[pl.ds(bH_mine, H), :],
                  out_ref.at[pl.ds(bH_mine, H), :], x_nbr)
        a2.wait_recv()
        a3.start()
        b2.wait_recv()
        b3.start()

        a3.wait_recv()
        b3.wait_recv()
        for r in (a0f, a0s, b0f, b0s, a1, b1, a2, b2, a3, b3):
            r.wait_send()

    return pl.pallas_call(
        body,
        out_shape=jax.ShapeDtypeStruct((M, N), jnp.float32),
        in_specs=[pl.BlockSpec(memory_space=pltpu.VMEM)],
        out_specs=pl.BlockSpec(memory_space=pltpu.VMEM),
        scratch_shapes=[
            pltpu.VMEM((H, N), jnp.float32),
            pltpu.VMEM((Q, N), jnp.float32),
            pltpu.VMEM((H, N), jnp.float32),
            pltpu.VMEM((Q, N), jnp.float32),
            pltpu.SemaphoreType.DMA((10,)),
            pltpu.SemaphoreType.DMA((10,)),
        ],
        compiler_params=pltpu.CompilerParams(collective_id=0),
    )(x)
